# baseline (speedup 1.0000x reference)
import functools

import jax
import jax.numpy as jnp
from jax import lax
from jax.experimental import pallas as pl
from jax.experimental.pallas import tpu as pltpu
from jax.experimental.pallas import tpu_sc as plsc

_NC = 2
_NS = 16
_NW = _NC * _NS

_TCOLS = 10
_CIDX = _TCOLS * 128


def kernel(x, emb_weight):
    n = x.shape[0]
    v, hidden = emb_weight.shape
    assert hidden == 16 and n % _CIDX == 0
    rowstride = 17
    emb_padded = jnp.pad(emb_weight, ((0, 0), (0, rowstride - hidden)))
    n_tc = n // 128
    n_chunks = n_tc // _TCOLS
    n_full = n_chunks // _NW
    n_rem = n_chunks - n_full * _NW
    assert n_full % 2 == 0

    mesh = plsc.VectorSubcoreMesh(core_axis_name="c", subcore_axis_name="s")

    @functools.partial(
        pl.kernel,
        out_type=jax.ShapeDtypeStruct((2, n_tc, 8, 128), jnp.float32),
        mesh=mesh,
        scratch_types=[
            pltpu.VMEM((v, rowstride), jnp.float32),
            pltpu.VMEM((2, _CIDX), jnp.int32),
            pltpu.VMEM((2, 2, _TCOLS, 8, 128), jnp.float32),
            pltpu.SemaphoreType.DMA,
            pltpu.SemaphoreType.DMA,
            pltpu.SemaphoreType.DMA,
            pltpu.SemaphoreType.DMA,
        ],
        compiler_params=pltpu.CompilerParams(
            use_tc_tiling_on_sc=False, needs_layout_passes=False),
    )
    def _run(x_hbm, tab_hbm, out_hbm, tab_v, idx_v, buf_v,
             sin0, sin1, sout0, sout1):
        wid = lax.axis_index("s") * _NC + lax.axis_index("c")
        pltpu.sync_copy(tab_hbm, tab_v)
        sin = (sin0, sin1)
        sout = (sout0, sout1)

        def idx_src(cid):
            return x_hbm.at[pl.ds(pl.multiple_of(cid * _CIDX, 8), _CIDX)]

        def out_dst(cid):
            return out_hbm.at[:, pl.ds(cid * _TCOLS, _TCOLS)]

        def gather_chunk(b):
            for tcb in range(_TCOLS):
                @plsc.parallel_loop(0, 8, unroll=4)
                def _gather(jj, _tcb=tcb, _b=b):
                    xv = idx_v[_b, pl.ds(_tcb * 128 + jj * 16, 16)]
                    vals = [plsc.load_gather(
                                tab_v, [xv, jnp.full((16,), h, jnp.int32)])
                            for h in range(16)]
                    for h in range(16):
                        buf_v[_b, h // 8, _tcb, h % 8, pl.ds(jj * 16, 16)] = vals[h]

        pltpu.async_copy(idx_src(wid), idx_v.at[0], sin[0])

        def pair(kp, carry):
            for b in (0, 1):
                kk = 2 * kp + b
                cid = wid + kk * _NW

                @pl.when(kk + 1 < n_full)
                def _prefetch():
                    nb = 1 - b
                    pltpu.async_copy(
                        idx_src(wid + (kk + 1) * _NW), idx_v.at[nb], sin[nb])

                pltpu.make_async_copy(idx_src(cid), idx_v.at[b], sin[b]).wait()

                @pl.when(kk >= 2)
                def _drain_prev():
                    pltpu.make_async_copy(
                        buf_v.at[b], out_dst(cid), sout[b]).wait()

                gather_chunk(b)
                pltpu.async_copy(buf_v.at[b], out_dst(cid), sout[b])
            return carry

        lax.fori_loop(0, n_full // 2, pair, 0, unroll=False)

        for b in (0, 1):
            cid = wid + (n_full - 2 + b) * _NW
            pltpu.make_async_copy(buf_v.at[b], out_dst(cid), sout[b]).wait()

        if n_rem:
            @pl.when(wid < n_rem)
            def _tail():
                cid = n_full * _NW + wid
                pltpu.sync_copy(idx_src(cid), idx_v.at[0])
                gather_chunk(0)
                pltpu.sync_copy(buf_v.at[0], out_dst(cid))

    out4 = _run(x.astype(jnp.int32), emb_padded)
    return out4.transpose(1, 3, 0, 2).reshape(n, hidden)


# speedup vs baseline: 1.0066x; 1.0066x over previous
import functools

import jax
import jax.numpy as jnp
from jax import lax
from jax.experimental import pallas as pl
from jax.experimental.pallas import tpu as pltpu
from jax.experimental.pallas import tpu_sc as plsc

_NC = 2
_NS = 16
_NW = _NC * _NS

_TCOLS = 10
_CIDX = _TCOLS * 128


def kernel(x, emb_weight):
    n = x.shape[0]
    v, hidden = emb_weight.shape
    assert hidden == 16 and n % _CIDX == 0
    rowstride = 17
    emb_padded = jnp.pad(emb_weight, ((0, 0), (0, rowstride - hidden)))
    n_tc = n // 128
    n_chunks = n_tc // _TCOLS
    n_full = n_chunks // _NW
    n_rem = n_chunks - n_full * _NW
    assert n_full % 2 == 0

    mesh = plsc.VectorSubcoreMesh(core_axis_name="c", subcore_axis_name="s")

    @functools.partial(
        pl.kernel,
        out_type=jax.ShapeDtypeStruct((2, n_tc, 8, 128), jnp.float32),
        mesh=mesh,
        scratch_types=[
            pltpu.VMEM((v, rowstride), jnp.float32),
            pltpu.VMEM((2, _CIDX), jnp.int32),
            pltpu.VMEM((2, 2, _TCOLS, 8, 128), jnp.float32),
            pltpu.SemaphoreType.DMA,
            pltpu.SemaphoreType.DMA,
            pltpu.SemaphoreType.DMA,
            pltpu.SemaphoreType.DMA,
        ],
        compiler_params=pltpu.CompilerParams(
            use_tc_tiling_on_sc=False, needs_layout_passes=False),
    )
    def _run(x_hbm, tab_hbm, out_hbm, tab_v, idx_v, buf_v,
             sin0, sin1, sout0, sout1):
        wid = lax.axis_index("s") * _NC + lax.axis_index("c")
        pltpu.sync_copy(tab_hbm, tab_v)
        sin = (sin0, sin1)
        sout = (sout0, sout1)

        def idx_src(cid):
            return x_hbm.at[pl.ds(pl.multiple_of(cid * _CIDX, 8), _CIDX)]

        def out_dst(cid):
            return out_hbm.at[:, pl.ds(cid * _TCOLS, _TCOLS)]

        def gather_chunk(b):
            for tcb in range(_TCOLS):
                @plsc.parallel_loop(0, 8, unroll=2)
                def _gather(jj, _tcb=tcb, _b=b):
                    xv = idx_v[_b, pl.ds(_tcb * 128 + jj * 16, 16)]
                    vals = [plsc.load_gather(
                                tab_v, [xv, jnp.full((16,), h, jnp.int32)])
                            for h in range(16)]
                    for h in range(16):
                        buf_v[_b, h // 8, _tcb, h % 8, pl.ds(jj * 16, 16)] = vals[h]

        pltpu.async_copy(idx_src(wid), idx_v.at[0], sin[0])

        def pair(kp, carry):
            for b in (0, 1):
                kk = 2 * kp + b
                cid = wid + kk * _NW

                @pl.when(kk + 1 < n_full)
                def _prefetch():
                    nb = 1 - b
                    pltpu.async_copy(
                        idx_src(wid + (kk + 1) * _NW), idx_v.at[nb], sin[nb])

                pltpu.make_async_copy(idx_src(cid), idx_v.at[b], sin[b]).wait()

                @pl.when(kk >= 2)
                def _drain_prev():
                    pltpu.make_async_copy(
                        buf_v.at[b], out_dst(cid), sout[b]).wait()

                gather_chunk(b)
                pltpu.async_copy(buf_v.at[b], out_dst(cid), sout[b])
            return carry

        lax.fori_loop(0, n_full // 2, pair, 0, unroll=False)

        for b in (0, 1):
            cid = wid + (n_full - 2 + b) * _NW
            pltpu.make_async_copy(buf_v.at[b], out_dst(cid), sout[b]).wait()

        if n_rem:
            @pl.when(wid < n_rem)
            def _tail():
                cid = n_full * _NW + wid
                pltpu.sync_copy(idx_src(cid), idx_v.at[0])
                gather_chunk(0)
                pltpu.sync_copy(buf_v.at[0], out_dst(cid))

    out4 = _run(x.astype(jnp.int32), emb_padded)
    return out4.transpose(1, 3, 0, 2).reshape(n, hidden)
